# baseline (speedup 1.0000x reference)
import functools

import jax
import jax.numpy as jnp
from jax import lax
from jax.experimental import pallas as pl
from jax.experimental.pallas import tpu as pltpu
from jax.experimental.pallas import tpu_sc as plsc

N_EXPERTS = 64
TOP_K = 2
D_MODEL = 1024
D_FF = 1024
N_TOKENS = 2048
N_PAIRS = N_TOKENS * TOP_K

BM = 64
_MC = 128
_NW = 32


def _meta_kernel(e_ref, ew_ref, inv_ref, starts_ref, ends_ref, ewa_ref,
                 ewb_ref):
  ones16 = jnp.ones((1, 16), jnp.float32)
  ewa_ref[...] = ew_ref[:, 0:1] * ones16
  ewb_ref[...] = ew_ref[:, 1:2] * ones16
  lanes = lax.broadcasted_iota(jnp.int32, (1, N_EXPERTS), 1)
  nch = N_PAIRS // _MC

  counts = jnp.zeros((1, N_EXPERTS), jnp.float32)
  for c in range(nch):
    ech = e_ref[c * _MC:(c + 1) * _MC, :]
    oh = (ech == lanes).astype(jnp.float32)
    counts = counts + jnp.sum(oh, axis=0, keepdims=True)

  r64 = lax.broadcasted_iota(jnp.int32, (N_EXPERTS, N_EXPERTS), 0)
  c64 = lax.broadcasted_iota(jnp.int32, (N_EXPERTS, N_EXPERTS), 1)
  tri_excl = (r64 < c64).astype(jnp.float32)
  starts = jnp.dot(counts, tri_excl, preferred_element_type=jnp.float32)

  rI = lax.broadcasted_iota(jnp.int32, (_MC, _MC), 0)
  cI = lax.broadcasted_iota(jnp.int32, (_MC, _MC), 1)
  tri_incl = (cI <= rI).astype(jnp.float32)

  carry = jnp.zeros((1, N_EXPERTS), jnp.float32)
  for c in range(nch):
    ech = e_ref[c * _MC:(c + 1) * _MC, :]
    oh = (ech == lanes).astype(jnp.float32)
    incl = jnp.dot(tri_incl, oh, preferred_element_type=jnp.float32)
    pos = (incl - oh) + starts + carry
    invch = jnp.sum(oh * pos, axis=1, keepdims=True)
    inv_ref[c * _MC:(c + 1) * _MC, :] = invch.astype(jnp.int32)
    carry = carry + jnp.sum(oh, axis=0, keepdims=True)

  starts_ref[...] = starts.astype(jnp.int32)
  ends_ref[...] = (starts + counts).astype(jnp.int32)


def _route_metadata(flat_e2d, ew):
  return pl.pallas_call(
      _meta_kernel,
      out_shape=(
          jax.ShapeDtypeStruct((N_PAIRS, 1), jnp.int32),
          jax.ShapeDtypeStruct((1, N_EXPERTS), jnp.int32),
          jax.ShapeDtypeStruct((1, N_EXPERTS), jnp.int32),
          jax.ShapeDtypeStruct((N_TOKENS, 16), jnp.float32),
          jax.ShapeDtypeStruct((N_TOKENS, 16), jnp.float32),
      ),
  )(flat_e2d, ew)


def _sc_order(inv_e, inv_o):
  mesh = plsc.VectorSubcoreMesh(core_axis_name="c", subcore_axis_name="s")
  tpw = N_TOKENS // _NW

  @functools.partial(
      pl.kernel,
      mesh=mesh,
      out_type=jax.ShapeDtypeStruct((N_PAIRS, 128), jnp.int32),
      scratch_types=[
          pltpu.VMEM((tpw,), jnp.int32),
          pltpu.VMEM((tpw,), jnp.int32),
          pltpu.VMEM((tpw, 128), jnp.int32),
          pltpu.SemaphoreType.DMA,
      ],
  )
  def k(ie_hbm, io_hbm, ord_hbm, i1_v, i2_v, tok_v, sem):
    wid = lax.axis_index("s") * 2 + lax.axis_index("c")
    t0 = wid * tpw
    m1 = pltpu.async_copy(ie_hbm.at[pl.ds(t0, tpw)], i1_v, sem)
    m2 = pltpu.async_copy(io_hbm.at[pl.ds(t0, tpw)], i2_v, sem)

    def fill(r, carry):
      for j in range(8):
        tok_v[r, pl.ds(j * 16, 16)] = jnp.zeros((16,), jnp.int32) + (t0 + r)
      return carry

    lax.fori_loop(0, tpw, fill, 0)
    m1.wait()
    m2.wait()
    c1 = pltpu.async_copy(tok_v, ord_hbm.at[i1_v], sem)
    c2 = pltpu.async_copy(tok_v, ord_hbm.at[i2_v], sem)
    c1.wait()
    c2.wait()

  return k(inv_e, inv_o)


_CCH = 16


def _sc_combine(b, inv_e, inv_o, ewa, ewb):
  mesh = plsc.VectorSubcoreMesh(core_axis_name="c", subcore_axis_name="s")
  tpw = N_TOKENS // _NW
  ncr = tpw // _CCH

  @functools.partial(
      pl.kernel,
      mesh=mesh,
      out_type=jax.ShapeDtypeStruct((N_TOKENS, D_MODEL), jnp.float32),
      scratch_types=[
          pltpu.VMEM((tpw,), jnp.int32),
          pltpu.VMEM((tpw,), jnp.int32),
          pltpu.VMEM((tpw, 16), jnp.float32),
          pltpu.VMEM((tpw, 16), jnp.float32),
          pltpu.VMEM((_CCH, D_MODEL), jnp.float32),
          pltpu.VMEM((_CCH, D_MODEL), jnp.float32),
          pltpu.VMEM((_CCH, D_MODEL), jnp.float32),
          pltpu.VMEM((_CCH, D_MODEL), jnp.float32),
          pltpu.SemaphoreType.DMA,
          pltpu.SemaphoreType.DMA,
          pltpu.SemaphoreType.DMA,
          pltpu.SemaphoreType.DMA,
          pltpu.SemaphoreType.DMA,
      ],
  )
  def k(b_hbm, ie_hbm, io_hbm, ewa_hbm, ewb_hbm, out_hbm, i1_v, i2_v, ewa_v,
        ewb_v, bufa0, bufb0, bufa1, bufb1, sg0, sg1, so0, so1, sidx):
    wid = lax.axis_index("s") * 2 + lax.axis_index("c")
    t00 = wid * tpw
    m1 = pltpu.async_copy(ie_hbm.at[pl.ds(t00, tpw)], i1_v, sidx)
    m2 = pltpu.async_copy(io_hbm.at[pl.ds(t00, tpw)], i2_v, sidx)
    m3 = pltpu.async_copy(ewa_hbm.at[pl.ds(t00, tpw)], ewa_v, sidx)
    m4 = pltpu.async_copy(ewb_hbm.at[pl.ds(t00, tpw)], ewb_v, sidx)
    m1.wait()
    m2.wait()
    m3.wait()
    m4.wait()

    bufa = (bufa0, bufa1)
    bufb = (bufb0, bufb1)
    sg = (sg0, sg1)
    so = (so0, so1)
    gathers = [None] * ncr
    outs = [None] * ncr

    def fire(c):
      slot = c % 2
      g1 = pltpu.async_copy(
          b_hbm.at[i1_v.at[pl.ds(c * _CCH, _CCH)]], bufa[slot], sg[slot])
      g2 = pltpu.async_copy(
          b_hbm.at[i2_v.at[pl.ds(c * _CCH, _CCH)]], bufb[slot], sg[slot])
      gathers[c] = (g1, g2)

    fire(0)
    for c in range(ncr):
      slot = c % 2
      t0 = t00 + c * _CCH
      if c >= 1:
        outs[c - 1].wait()
      if c + 1 < ncr:
        fire(c + 1)
      g1, g2 = gathers[c]
      g1.wait()
      g2.wait()

      ba, bb = bufa[slot], bufb[slot]

      def addrow(r, carry, ba=ba, bb=bb, c=c):
        wa = ewa_v[c * _CCH + r, pl.ds(0, 16)]
        wb = ewb_v[c * _CCH + r, pl.ds(0, 16)]
        for j in range(D_MODEL // 16):
          s = pl.ds(j * 16, 16)
          ba[r, s] = wa * ba[r, s] + wb * bb[r, s]
        return carry

      lax.fori_loop(0, _CCH, addrow, 0)
      outs[c] = pltpu.async_copy(ba, out_hbm.at[pl.ds(t0, _CCH)], so[slot])
    outs[ncr - 1].wait()

  return k(b, inv_e, inv_o, ewa, ewb)


def _moe_mm_kernel(starts_ref, ends_ref, tok_ref, x_ref, w1_ref, w2_ref,
                   w3_ref, b_ref, a_scr):
  e = pl.program_id(0)
  start = starts_ref[e]
  end = ends_ref[e]
  astart = (start // 8) * 8
  nch = lax.div(end - astart + BM - 1, BM)

  def body(t, carry):
    base = jnp.minimum(astart + t * BM, N_PAIRS - BM)
    base = pl.multiple_of(base, 8)

    def grow(r, carry2):
      tid = tok_ref[base + r]
      a_scr[pl.ds(r, 1), :] = x_ref[pl.ds(tid, 1), :]
      return carry2

    lax.fori_loop(0, BM, grow, 0, unroll=8)
    a = a_scr[...]
    g = jnp.dot(a, w1_ref[0], preferred_element_type=jnp.float32)
    v = jnp.dot(a, w2_ref[0], preferred_element_type=jnp.float32)
    h = g * jax.nn.sigmoid(g) * v
    o = jnp.dot(h, w3_ref[0], preferred_element_type=jnp.float32)
    rows = base + lax.broadcasted_iota(jnp.int32, (BM, 1), 0)
    mask = (rows >= start) & (rows < end)
    cur = b_ref[pl.ds(base, BM), :]
    b_ref[pl.ds(base, BM), :] = jnp.where(mask, o, cur)
    return carry

  lax.fori_loop(0, nch, body, 0)


def _grouped_swiglu(starts, ends, tok, x, w1, w2, w3):
  grid_spec = pltpu.PrefetchScalarGridSpec(
      num_scalar_prefetch=3,
      grid=(N_EXPERTS,),
      in_specs=[
          pl.BlockSpec((N_TOKENS, D_MODEL), lambda e, *_: (0, 0)),
          pl.BlockSpec((1, D_MODEL, D_FF), lambda e, *_: (e, 0, 0)),
          pl.BlockSpec((1, D_MODEL, D_FF), lambda e, *_: (e, 0, 0)),
          pl.BlockSpec((1, D_FF, D_MODEL), lambda e, *_: (e, 0, 0)),
      ],
      out_specs=pl.BlockSpec((N_PAIRS, D_MODEL), lambda e, *_: (0, 0)),
      scratch_shapes=[pltpu.VMEM((BM, D_MODEL), jnp.float32)],
  )
  return pl.pallas_call(
      _moe_mm_kernel,
      grid_spec=grid_spec,
      out_shape=jax.ShapeDtypeStruct((N_PAIRS, D_MODEL), jnp.float32),
      compiler_params=pltpu.CompilerParams(
          dimension_semantics=("arbitrary",)),
  )(starts, ends, tok, x, w1, w2, w3)


def kernel(x, expert_indices, expert_weights, w1, w2, w3):
  flat_e2d = expert_indices.reshape(N_PAIRS, 1)
  inv, starts, ends, ewa, ewb = _route_metadata(flat_e2d, expert_weights)
  inv2 = inv.reshape(N_TOKENS, TOP_K)
  inv_e = inv2[:, 0]
  inv_o = inv2[:, 1]

  order16 = _sc_order(inv_e, inv_o)
  tok = order16[:, 0]
  b = _grouped_swiglu(starts.reshape(-1), ends.reshape(-1), tok, x, w1, w2,
                      w3)
  out = _sc_combine(b, inv_e, inv_o, ewa, ewb)
  return out


# speedup vs baseline: 1.0450x; 1.0450x over previous
import functools

import jax
import jax.numpy as jnp
from jax import lax
from jax.experimental import pallas as pl
from jax.experimental.pallas import tpu as pltpu
from jax.experimental.pallas import tpu_sc as plsc

N_EXPERTS = 64
TOP_K = 2
D_MODEL = 1024
D_FF = 1024
N_TOKENS = 2048
N_PAIRS = N_TOKENS * TOP_K

BM = 128
_MC = 128
_NW = 32


def _meta_kernel(ei_ref, ew_ref, inve_ref, invo_ref, starts_ref, ends_ref,
                 ewa_ref, ewb_ref):
  ones16 = jnp.ones((1, 16), jnp.float32)
  ewa_ref[...] = ew_ref[:, 0:1] * ones16
  ewb_ref[...] = ew_ref[:, 1:2] * ones16

  lanes = lax.broadcasted_iota(jnp.int32, (1, N_EXPERTS), 1)
  nch = N_TOKENS // _MC

  def onehot(c, k):
    ech = ei_ref[c * _MC:(c + 1) * _MC, k:k + 1]
    return (ech == lanes).astype(jnp.float32)

  cnts = []
  for c in range(nch):
    for k in (0, 1):
      cnts.append(jnp.sum(onehot(c, k), axis=0, keepdims=True))
  carries = []
  acc = jnp.zeros((1, N_EXPERTS), jnp.float32)
  for s in range(2 * nch):
    carries.append(acc)
    acc = acc + cnts[s]
  counts = acc

  r64 = lax.broadcasted_iota(jnp.int32, (N_EXPERTS, N_EXPERTS), 0)
  c64 = lax.broadcasted_iota(jnp.int32, (N_EXPERTS, N_EXPERTS), 1)
  tri_excl = (r64 < c64).astype(jnp.float32)
  starts = jnp.dot(counts, tri_excl, preferred_element_type=jnp.float32)

  rI = lax.broadcasted_iota(jnp.int32, (_MC, _MC), 0)
  cI = lax.broadcasted_iota(jnp.int32, (_MC, _MC), 1)
  tri_incl = (cI <= rI).astype(jnp.float32)
  ones_e1 = jnp.ones((N_EXPERTS, 1), jnp.float32)

  s = 0
  for c in range(nch):
    for k in (0, 1):
      oh = onehot(c, k)
      incl = jnp.dot(tri_incl, oh, preferred_element_type=jnp.float32)
      pos = (incl - oh) + starts + carries[s]
      invch = jnp.dot(oh * pos, ones_e1, preferred_element_type=jnp.float32)
      dst = inve_ref if k == 0 else invo_ref
      dst[c * _MC:(c + 1) * _MC, :] = invch.astype(jnp.int32)
      s += 1

  starts_ref[...] = starts.astype(jnp.int32)
  ends_ref[...] = (starts + counts).astype(jnp.int32)


def _route_metadata(ei, ew):
  return pl.pallas_call(
      _meta_kernel,
      out_shape=(
          jax.ShapeDtypeStruct((N_TOKENS, 1), jnp.int32),
          jax.ShapeDtypeStruct((N_TOKENS, 1), jnp.int32),
          jax.ShapeDtypeStruct((1, N_EXPERTS), jnp.int32),
          jax.ShapeDtypeStruct((1, N_EXPERTS), jnp.int32),
          jax.ShapeDtypeStruct((N_TOKENS, 16), jnp.float32),
          jax.ShapeDtypeStruct((N_TOKENS, 16), jnp.float32),
      ),
  )(ei, ew)


def _sc_scatter(x, inv_e, inv_o):
  mesh = plsc.VectorSubcoreMesh(core_axis_name="c", subcore_axis_name="s")
  tpw = N_TOKENS // _NW

  @functools.partial(
      pl.kernel,
      mesh=mesh,
      out_type=jax.ShapeDtypeStruct((N_PAIRS, D_MODEL), jnp.float32),
      scratch_types=[
          pltpu.VMEM((tpw,), jnp.int32),
          pltpu.VMEM((tpw,), jnp.int32),
          pltpu.VMEM((tpw, D_MODEL), jnp.float32),
          pltpu.SemaphoreType.DMA,
      ],
  )
  def k(x_hbm, ie_hbm, io_hbm, a_hbm, i1_v, i2_v, rows_v, sem):
    wid = lax.axis_index("s") * 2 + lax.axis_index("c")
    t0 = wid * tpw
    m1 = pltpu.async_copy(ie_hbm.at[pl.ds(t0, tpw)], i1_v, sem)
    m2 = pltpu.async_copy(io_hbm.at[pl.ds(t0, tpw)], i2_v, sem)
    m3 = pltpu.async_copy(x_hbm.at[pl.ds(t0, tpw)], rows_v, sem)
    m1.wait()
    m2.wait()
    m3.wait()
    c1 = pltpu.async_copy(rows_v, a_hbm.at[i1_v], sem)
    c2 = pltpu.async_copy(rows_v, a_hbm.at[i2_v], sem)
    c1.wait()
    c2.wait()

  return k(x, inv_e, inv_o)


_CCH = 16


def _sc_combine(b, inv_e, inv_o, ewa, ewb):
  mesh = plsc.VectorSubcoreMesh(core_axis_name="c", subcore_axis_name="s")
  tpw = N_TOKENS // _NW
  ncr = tpw // _CCH

  @functools.partial(
      pl.kernel,
      mesh=mesh,
      out_type=jax.ShapeDtypeStruct((N_TOKENS, D_MODEL), jnp.float32),
      scratch_types=[
          pltpu.VMEM((tpw,), jnp.int32),
          pltpu.VMEM((tpw,), jnp.int32),
          pltpu.VMEM((tpw, 16), jnp.float32),
          pltpu.VMEM((tpw, 16), jnp.float32),
          pltpu.VMEM((_CCH, D_MODEL), jnp.float32),
          pltpu.VMEM((_CCH, D_MODEL), jnp.float32),
          pltpu.VMEM((_CCH, D_MODEL), jnp.float32),
          pltpu.VMEM((_CCH, D_MODEL), jnp.float32),
          pltpu.SemaphoreType.DMA,
          pltpu.SemaphoreType.DMA,
          pltpu.SemaphoreType.DMA,
          pltpu.SemaphoreType.DMA,
          pltpu.SemaphoreType.DMA,
      ],
  )
  def k(b_hbm, ie_hbm, io_hbm, ewa_hbm, ewb_hbm, out_hbm, i1_v, i2_v, ewa_v,
        ewb_v, bufa0, bufb0, bufa1, bufb1, sg0, sg1, so0, so1, sidx):
    wid = lax.axis_index("s") * 2 + lax.axis_index("c")
    t00 = wid * tpw
    m1 = pltpu.async_copy(ie_hbm.at[pl.ds(t00, tpw)], i1_v, sidx)
    m2 = pltpu.async_copy(io_hbm.at[pl.ds(t00, tpw)], i2_v, sidx)
    m3 = pltpu.async_copy(ewa_hbm.at[pl.ds(t00, tpw)], ewa_v, sidx)
    m4 = pltpu.async_copy(ewb_hbm.at[pl.ds(t00, tpw)], ewb_v, sidx)
    m1.wait()
    m2.wait()
    m3.wait()
    m4.wait()

    bufa = (bufa0, bufa1)
    bufb = (bufb0, bufb1)
    sg = (sg0, sg1)
    so = (so0, so1)
    gathers = [None] * ncr
    outs = [None] * ncr

    def fire(c):
      slot = c % 2
      g1 = pltpu.async_copy(
          b_hbm.at[i1_v.at[pl.ds(c * _CCH, _CCH)]], bufa[slot], sg[slot])
      g2 = pltpu.async_copy(
          b_hbm.at[i2_v.at[pl.ds(c * _CCH, _CCH)]], bufb[slot], sg[slot])
      gathers[c] = (g1, g2)

    fire(0)
    for c in range(ncr):
      slot = c % 2
      t0 = t00 + c * _CCH
      if c >= 1:
        outs[c - 1].wait()
      if c + 1 < ncr:
        fire(c + 1)
      g1, g2 = gathers[c]
      g1.wait()
      g2.wait()

      ba, bb = bufa[slot], bufb[slot]

      def addrow(r, carry, ba=ba, bb=bb, c=c):
        wa = ewa_v[c * _CCH + r, pl.ds(0, 16)]
        wb = ewb_v[c * _CCH + r, pl.ds(0, 16)]
        for j in range(D_MODEL // 16):
          s = pl.ds(j * 16, 16)
          ba[r, s] = wa * ba[r, s] + wb * bb[r, s]
        return carry

      lax.fori_loop(0, _CCH, addrow, 0)
      outs[c] = pltpu.async_copy(ba, out_hbm.at[pl.ds(t0, _CCH)], so[slot])
    outs[ncr - 1].wait()

  return k(b, inv_e, inv_o, ewa, ewb)


def _moe_mm_kernel(starts_ref, ends_ref, a_ref, w1_ref, w2_ref, w3_ref,
                   b_ref):
  e = pl.program_id(0)
  start = starts_ref[e]
  end = ends_ref[e]
  astart = (start // 8) * 8
  nch = lax.div(end - astart + BM - 1, BM)

  def body(t, carry):
    base = jnp.minimum(astart + t * BM, N_PAIRS - BM)
    base = pl.multiple_of(base, 8)
    a = a_ref[pl.ds(base, BM), :]
    g = jnp.dot(a, w1_ref[0], preferred_element_type=jnp.float32)
    v = jnp.dot(a, w2_ref[0], preferred_element_type=jnp.float32)
    h = g * jax.nn.sigmoid(g) * v
    o = jnp.dot(h, w3_ref[0], preferred_element_type=jnp.float32)
    rows = base + lax.broadcasted_iota(jnp.int32, (BM, 1), 0)
    mask = (rows >= start) & (rows < end)
    cur = b_ref[pl.ds(base, BM), :]
    b_ref[pl.ds(base, BM), :] = jnp.where(mask, o, cur)
    return carry

  lax.fori_loop(0, nch, body, 0)


def _grouped_swiglu(starts, ends, a, w1, w2, w3):
  grid_spec = pltpu.PrefetchScalarGridSpec(
      num_scalar_prefetch=2,
      grid=(N_EXPERTS,),
      in_specs=[
          pl.BlockSpec((N_PAIRS, D_MODEL), lambda e, *_: (0, 0)),
          pl.BlockSpec((1, D_MODEL, D_FF), lambda e, *_: (e, 0, 0)),
          pl.BlockSpec((1, D_MODEL, D_FF), lambda e, *_: (e, 0, 0)),
          pl.BlockSpec((1, D_FF, D_MODEL), lambda e, *_: (e, 0, 0)),
      ],
      out_specs=pl.BlockSpec((N_PAIRS, D_MODEL), lambda e, *_: (0, 0)),
  )
  return pl.pallas_call(
      _moe_mm_kernel,
      grid_spec=grid_spec,
      out_shape=jax.ShapeDtypeStruct((N_PAIRS, D_MODEL), jnp.float32),
      compiler_params=pltpu.CompilerParams(
          dimension_semantics=("arbitrary",)),
  )(starts, ends, a, w1, w2, w3)


def kernel(x, expert_indices, expert_weights, w1, w2, w3):
  inv_e2, inv_o2, starts, ends, ewa, ewb = _route_metadata(
      expert_indices, expert_weights)
  inv_e = inv_e2.reshape(-1)
  inv_o = inv_o2.reshape(-1)

  a = _sc_scatter(x, inv_e, inv_o)
  b = _grouped_swiglu(starts.reshape(-1), ends.reshape(-1), a, w1, w2, w3)
  out = _sc_combine(b, inv_e, inv_o, ewa, ewb)
  return out
